# baseline (speedup 1.0000x reference)
import jax
import jax.numpy as jnp
from jax.experimental import pallas as pl
from jax.experimental.pallas import tpu as pltpu

_BM = 1024
_BN = 1024


def _matmul_body(x_ref, w_ref, b_ref, o_ref, w_bf, x_bf):
    i = pl.program_id(0)
    j = pl.program_id(1)

    @pl.when((i == 0) & (j == 0))
    def _cast_w():
        w = w_ref[...].astype(jnp.bfloat16)
        w_bf[0] = w[:, :_BN]
        w_bf[1] = w[:, _BN:]

    @pl.when(j == 0)
    def _cast_x():
        x_bf[...] = x_ref[...].astype(jnp.bfloat16)

    o_ref[...] = (
        jnp.dot(x_bf[...], w_bf[j], preferred_element_type=jnp.float32)
        + b_ref[...]
    )


def kernel(inputs, W, b):
    m, k = inputs.shape
    n = W.shape[1]
    b2 = b.reshape(1, n)
    grid = (m // _BM, n // _BN)
    return pl.pallas_call(
        _matmul_body,
        grid=grid,
        in_specs=[
            pl.BlockSpec((_BM, k), lambda i, j: (i, 0)),
            pl.BlockSpec((k, n), lambda i, j: (0, 0)),
            pl.BlockSpec((1, _BN), lambda i, j: (0, j)),
        ],
        out_specs=pl.BlockSpec((_BM, _BN), lambda i, j: (i, j)),
        out_shape=jax.ShapeDtypeStruct((m, n), jnp.float32),
        scratch_shapes=[
            pltpu.VMEM((2, k, _BN), jnp.bfloat16),
            pltpu.VMEM((_BM, k), jnp.bfloat16),
        ],
        compiler_params=pltpu.CompilerParams(
            dimension_semantics=("arbitrary", "arbitrary"),
        ),
    )(inputs, W, b2)


# speedup vs baseline: 1.1155x; 1.1155x over previous
import jax
import jax.numpy as jnp
from jax.experimental import pallas as pl
from jax.experimental.pallas import tpu as pltpu

_BM = 1024


def _matmul_body(x_ref, w_ref, b_ref, o_ref):
    o_ref[...] = (
        jax.lax.dot_general(
            x_ref[...],
            w_ref[...],
            dimension_numbers=(((1,), (0,)), ((), ())),
            precision=jax.lax.Precision.DEFAULT,
            preferred_element_type=jnp.float32,
        )
        + b_ref[...]
    )


def kernel(inputs, W, b):
    m, k = inputs.shape
    n = W.shape[1]
    b2 = b.reshape(1, n)
    grid = (m // _BM,)
    return pl.pallas_call(
        _matmul_body,
        grid=grid,
        in_specs=[
            pl.BlockSpec((_BM, k), lambda i: (i, 0)),
            pl.BlockSpec((k, n), lambda i: (0, 0)),
            pl.BlockSpec((1, n), lambda i: (0, 0)),
        ],
        out_specs=pl.BlockSpec((_BM, n), lambda i: (i, 0)),
        out_shape=jax.ShapeDtypeStruct((m, n), jnp.float32),
        compiler_params=pltpu.CompilerParams(
            dimension_semantics=("arbitrary",),
        ),
    )(inputs, W, b2)
